# speedup vs baseline: 360.4374x; 360.4374x over previous
import functools

import jax
import jax.numpy as jnp
from jax.experimental import pallas as pl
from jax.experimental.pallas import tpu as pltpu

N_HEADS = 12
WINDOW = 64
TAU = 2.0
NEG = -1e30


def _rank_kernel(col_full, row_blk, row_full, col_blk, rank_row, rank_col, *, T, BR):
    j0 = pl.program_id(1) * BR
    ck_col = col_full[0, :, :]
    cj_row = row_blk[0, :, :]
    k_col = jax.lax.broadcasted_iota(jnp.int32, (T, 1), 0)
    j_row = j0 + jax.lax.broadcasted_iota(jnp.int32, (1, BR), 1)
    lt = ck_col < cj_row
    eq = (ck_col == cj_row) & (k_col < j_row)
    rank_row[0, 0, :] = jnp.sum((lt | eq).astype(jnp.int32), axis=0)
    ck_row = row_full[0, :, :]
    cj_col = col_blk[0, :, :]
    k_row = jax.lax.broadcasted_iota(jnp.int32, (1, T), 1)
    j_col = j0 + jax.lax.broadcasted_iota(jnp.int32, (BR, 1), 0)
    lt2 = ck_row < cj_col
    eq2 = (ck_row == cj_col) & (k_row < j_col)
    rank_col[0, :, 0] = jnp.sum((lt2 | eq2).astype(jnp.int32), axis=1)


def _gather_qkv_kernel(rank_row, h_full, coord_row, w, bias, qkv_out, cs_out,
                       *, T, BS):
    qs = pl.program_id(1) * BS
    rk = rank_row[0, :, :]
    tgt = qs + jax.lax.broadcasted_iota(jnp.int32, (BS, 1), 0)
    P = (rk == tgt).astype(jnp.float32)
    hs = jnp.dot(P, h_full[0], preferred_element_type=jnp.float32)
    qkv = jnp.dot(hs, w[:, :], preferred_element_type=jnp.float32)
    qkv_out[0, :, :] = qkv + bias[0, :]
    cs_out[0, :, :] = jnp.sum(P * coord_row[0, :, :], axis=1, keepdims=True)


def _attn_kernel(depot_ref, q_ref, k_ref, v_ref, ct_ref, cu_ref, out_ref,
                 *, T, BQ, H, DH):
    b = pl.program_id(0)
    qs = pl.program_id(1) * BQ
    d = depot_ref[b]
    scale = 1.0 / (DH ** 0.5)
    inv_tau = 1.0 / TAU
    ct = ct_ref[0, :, :]
    cu = cu_ref[0, :, :]
    t = qs + jax.lax.broadcasted_iota(jnp.int32, (BQ, 1), 0)
    u = jax.lax.broadcasted_iota(jnp.int32, (1, T), 1)
    half = WINDOW // 2
    start = jnp.clip(t - half, 0, T - WINDOW)
    mask = ((u >= start) & (u < start + WINDOW)) | (u == d) | (t == d)
    q_extra = jnp.concatenate(
        [-inv_tau * ct * ct, (2.0 * inv_tau) * ct,
         jnp.full((BQ, 1), -inv_tau, jnp.float32)], axis=1)
    k_extra = jnp.concatenate(
        [jnp.ones((T, 1), jnp.float32), cu, cu * cu], axis=1)
    for h in range(H):
        lo, hi = h * DH, (h + 1) * DH
        qa = jnp.concatenate([q_ref[0, :, lo:hi] * scale, q_extra], axis=1)
        ka = jnp.concatenate([k_ref[0, :, lo:hi], k_extra], axis=1)
        s = jax.lax.dot_general(qa, ka, (((1,), (1,)), ((), ())),
                                preferred_element_type=jnp.float32)
        s = jnp.where(mask, s, NEG)
        mx = jnp.max(s, axis=1, keepdims=True)
        p = jnp.where(mask, jnp.exp(s - mx), 0.0)
        dn = jnp.sum(p, axis=1, keepdims=True)
        ctx = jnp.dot(p, v_ref[0, :, lo:hi],
                      preferred_element_type=jnp.float32) / dn
        out_ref[0, :, lo:hi] = ctx


def _unsort_proj_kernel(rank_col, ctx_full, w, bias, out_ref, *, T, BS):
    rk = rank_col[0, :, :]
    u = jax.lax.broadcasted_iota(jnp.int32, (1, T), 1)
    G = (rk == u).astype(jnp.float32)
    y = jnp.dot(G, ctx_full[0], preferred_element_type=jnp.float32)
    out_ref[0, :, :] = jnp.dot(y, w[:, :],
                               preferred_element_type=jnp.float32) + bias[0, :]


def kernel(h, coord_1d, Wq_w, Wq_b, Wk_w, Wk_b, Wv_w, Wv_b, Wo_w, Wo_b):
    B, T, E = h.shape
    H = N_HEADS
    DH = E // H
    BR = 256
    BS = 256
    BQ = 256

    coord_row = coord_1d.reshape(B, 1, T)
    coord_col = coord_1d.reshape(B, T, 1)
    w_qkv = jnp.concatenate([Wq_w, Wk_w, Wv_w], axis=1)
    b_qkv = jnp.concatenate([Wq_b, Wk_b, Wv_b]).reshape(1, 3 * E)
    b_o = Wo_b.reshape(1, E)

    rank_row, rank_col = pl.pallas_call(
        functools.partial(_rank_kernel, T=T, BR=BR),
        grid=(B, T // BR),
        in_specs=[
            pl.BlockSpec((1, T, 1), lambda b, j: (b, 0, 0)),
            pl.BlockSpec((1, 1, BR), lambda b, j: (b, 0, j)),
            pl.BlockSpec((1, 1, T), lambda b, j: (b, 0, 0)),
            pl.BlockSpec((1, BR, 1), lambda b, j: (b, j, 0)),
        ],
        out_specs=[
            pl.BlockSpec((1, 1, BR), lambda b, j: (b, 0, j)),
            pl.BlockSpec((1, BR, 1), lambda b, j: (b, j, 0)),
        ],
        out_shape=[
            jax.ShapeDtypeStruct((B, 1, T), jnp.int32),
            jax.ShapeDtypeStruct((B, T, 1), jnp.int32),
        ],
    )(coord_col, coord_row, coord_row, coord_col)

    depot = rank_row[:, 0, 0]

    qkv, cs_col = pl.pallas_call(
        functools.partial(_gather_qkv_kernel, T=T, BS=BS),
        grid=(B, T // BS),
        in_specs=[
            pl.BlockSpec((1, 1, T), lambda b, i: (b, 0, 0)),
            pl.BlockSpec((1, T, E), lambda b, i: (b, 0, 0)),
            pl.BlockSpec((1, 1, T), lambda b, i: (b, 0, 0)),
            pl.BlockSpec((E, 3 * E), lambda b, i: (0, 0)),
            pl.BlockSpec((1, 3 * E), lambda b, i: (0, 0)),
        ],
        out_specs=[
            pl.BlockSpec((1, BS, 3 * E), lambda b, i: (b, i, 0)),
            pl.BlockSpec((1, BS, 1), lambda b, i: (b, i, 0)),
        ],
        out_shape=[
            jax.ShapeDtypeStruct((B, T, 3 * E), jnp.float32),
            jax.ShapeDtypeStruct((B, T, 1), jnp.float32),
        ],
    )(rank_row, h, coord_row, w_qkv, b_qkv)

    ctx = pl.pallas_call(
        functools.partial(_attn_kernel, T=T, BQ=BQ, H=H, DH=DH),
        grid_spec=pltpu.PrefetchScalarGridSpec(
            num_scalar_prefetch=1,
            grid=(B, T // BQ),
            in_specs=[
                pl.BlockSpec((1, BQ, E), lambda b, i, dref: (b, i, 0)),
                pl.BlockSpec((1, T, E), lambda b, i, dref: (b, 0, 1)),
                pl.BlockSpec((1, T, E), lambda b, i, dref: (b, 0, 2)),
                pl.BlockSpec((1, BQ, 1), lambda b, i, dref: (b, i, 0)),
                pl.BlockSpec((1, T, 1), lambda b, i, dref: (b, 0, 0)),
            ],
            out_specs=pl.BlockSpec((1, BQ, E), lambda b, i, dref: (b, i, 0)),
        ),
        out_shape=jax.ShapeDtypeStruct((B, T, E), jnp.float32),
    )(depot, qkv, qkv, qkv, cs_col, cs_col)

    out = pl.pallas_call(
        functools.partial(_unsort_proj_kernel, T=T, BS=BS),
        grid=(B, T // BS),
        in_specs=[
            pl.BlockSpec((1, BS, 1), lambda b, i: (b, i, 0)),
            pl.BlockSpec((1, T, E), lambda b, i: (b, 0, 0)),
            pl.BlockSpec((E, E), lambda b, i: (0, 0)),
            pl.BlockSpec((1, E), lambda b, i: (0, 0)),
        ],
        out_specs=pl.BlockSpec((1, BS, E), lambda b, i: (b, i, 0)),
        out_shape=jax.ShapeDtypeStruct((B, T, E), jnp.float32),
    )(rank_col, ctx, Wo_w, b_o)

    return out


# speedup vs baseline: 425.3180x; 1.1800x over previous
import functools

import jax
import jax.numpy as jnp
from jax.experimental import pallas as pl
from jax.experimental.pallas import tpu as pltpu

N_HEADS = 12
WINDOW = 64
TAU = 2.0
NEG = -1e30


def _rank_kernel(col_full, row_blk, row_full, col_blk, rank_row, rank_col, *, T, BR):
    j0 = pl.program_id(1) * BR
    ck_col = col_full[0, :, :]
    cj_row = row_blk[0, :, :]
    k_col = jax.lax.broadcasted_iota(jnp.int32, (T, 1), 0)
    j_row = j0 + jax.lax.broadcasted_iota(jnp.int32, (1, BR), 1)
    lt = ck_col < cj_row
    eq = (ck_col == cj_row) & (k_col < j_row)
    rank_row[0, 0, :] = jnp.sum((lt | eq).astype(jnp.int32), axis=0)
    ck_row = row_full[0, :, :]
    cj_col = col_blk[0, :, :]
    k_row = jax.lax.broadcasted_iota(jnp.int32, (1, T), 1)
    j_col = j0 + jax.lax.broadcasted_iota(jnp.int32, (BR, 1), 0)
    lt2 = ck_row < cj_col
    eq2 = (ck_row == cj_col) & (k_row < j_col)
    rank_col[0, :, 0] = jnp.sum((lt2 | eq2).astype(jnp.int32), axis=1)


def _gather_qkv_kernel(rank_row, h_full, coord_row, w, bias, qkv_out, cs_out,
                       *, T, BS):
    qs = pl.program_id(1) * BS
    rk = rank_row[0, :, :]
    tgt = qs + jax.lax.broadcasted_iota(jnp.int32, (BS, 1), 0)
    P = (rk == tgt).astype(jnp.float32)
    hs = jnp.dot(P, h_full[0], preferred_element_type=jnp.float32)
    qkv = jnp.dot(hs, w[:, :], preferred_element_type=jnp.float32)
    qkv_out[0, :, :] = qkv + bias[0, :]
    cs_out[0, :, :] = jnp.sum(P * coord_row[0, :, :], axis=1, keepdims=True)


def _dyn_row(ref, idx, lo, hi):
    base = pl.multiple_of((idx // 8) * 8, 8)
    blk = ref[0, pl.ds(base, 8), lo:hi]
    sel = jax.lax.broadcasted_iota(jnp.int32, (8, 1), 0) == (idx - base)
    return jnp.sum(jnp.where(sel, blk, 0.0), axis=0, keepdims=True)


def _attn_kernel(depot_ref, q_ref, k_ref, v_ref, ct_ref, cu_ref, out_ref,
                 *, T, BQ, H, DH):
    b = pl.program_id(0)
    qs = pl.program_id(1) * BQ
    d = depot_ref[b]
    scale = 1.0 / (DH ** 0.5)
    inv_tau = 1.0 / TAU
    BK = BQ + WINDOW
    half = WINDOW // 2
    h0 = jnp.clip(qs - half, 0, T - BK)
    h0 = pl.multiple_of(h0, 32)
    ct = ct_ref[0, :, :]
    cu = cu_ref[0, pl.ds(h0, BK), :]
    cu_full = cu_ref[0, :, :]
    cd = _dyn_row(cu_ref, d, 0, 1)
    t = qs + jax.lax.broadcasted_iota(jnp.int32, (BQ, 1), 0)
    u = h0 + jax.lax.broadcasted_iota(jnp.int32, (1, BK), 1)
    start = jnp.clip(t - half, 0, T - WINDOW)
    mask = (u >= start) & (u < start + WINDOW)
    keep_d = ~((d >= start) & (d < start + WINDOW))
    is_d = t == d
    q_extra = jnp.concatenate(
        [-inv_tau * ct * ct, (2.0 * inv_tau) * ct,
         jnp.full((BQ, 1), -inv_tau, jnp.float32)], axis=1)
    qd_extra = jnp.concatenate(
        [-inv_tau * cd * cd, (2.0 * inv_tau) * cd,
         jnp.full((1, 1), -inv_tau, jnp.float32)], axis=1)
    k_extra = jnp.concatenate(
        [jnp.ones((BK, 1), jnp.float32), cu, cu * cu], axis=1)
    k_extra_full = jnp.concatenate(
        [jnp.ones((T, 1), jnp.float32), cu_full, cu_full * cu_full],
        axis=1)
    kd_extra = jnp.concatenate(
        [jnp.ones((1, 1), jnp.float32), cd, cd * cd], axis=1)
    dq = jnp.clip(d - qs, 0, BQ - 1)
    for h in range(H):
        lo, hi = h * DH, (h + 1) * DH
        qa = jnp.concatenate([q_ref[0, :, lo:hi] * scale, q_extra], axis=1)
        kh = k_ref[0, pl.ds(h0, BK), lo:hi]
        vh = v_ref[0, pl.ds(h0, BK), lo:hi]
        ka = jnp.concatenate([kh, k_extra], axis=1)
        s = jax.lax.dot_general(qa, ka, (((1,), (1,)), ((), ())),
                                preferred_element_type=jnp.float32)
        s = jnp.where(mask, s, NEG)
        kd = jnp.concatenate([_dyn_row(k_ref, d, lo, hi), kd_extra], axis=1)
        sd = jax.lax.dot_general(qa, kd, (((1,), (1,)), ((), ())),
                                 preferred_element_type=jnp.float32)
        sd = jnp.where(keep_d, sd, NEG)
        mx = jnp.maximum(jnp.max(s, axis=1, keepdims=True), sd)
        p = jnp.where(mask, jnp.exp(s - mx), 0.0)
        pd = jnp.where(keep_d, jnp.exp(sd - mx), 0.0)
        dn = jnp.sum(p, axis=1, keepdims=True) + pd
        vd = _dyn_row(v_ref, d, lo, hi)
        ctx = (jnp.dot(p, vh, preferred_element_type=jnp.float32)
               + pd * vd) / dn
        qda = jnp.concatenate(
            [_dyn_row(q_ref, dq, lo, hi) * scale, qd_extra], axis=1)
        ka_full = jnp.concatenate([k_ref[0, :, lo:hi], k_extra_full], axis=1)
        sfull = jax.lax.dot_general(qda, ka_full, (((1,), (1,)), ((), ())),
                                    preferred_element_type=jnp.float32)
        mxf = jnp.max(sfull, axis=1, keepdims=True)
        pf = jnp.exp(sfull - mxf)
        ctx_d = (jnp.dot(pf, v_ref[0, :, lo:hi],
                         preferred_element_type=jnp.float32)
                 / jnp.sum(pf, axis=1, keepdims=True))
        out_ref[0, :, lo:hi] = jnp.where(is_d, ctx_d, ctx)


def _unsort_proj_kernel(rank_col, ctx_full, w, bias, out_ref, *, T, BS):
    rk = rank_col[0, :, :]
    u = jax.lax.broadcasted_iota(jnp.int32, (1, T), 1)
    G = (rk == u).astype(jnp.float32)
    y = jnp.dot(G, ctx_full[0], preferred_element_type=jnp.float32)
    out_ref[0, :, :] = jnp.dot(y, w[:, :],
                               preferred_element_type=jnp.float32) + bias[0, :]


def kernel(h, coord_1d, Wq_w, Wq_b, Wk_w, Wk_b, Wv_w, Wv_b, Wo_w, Wo_b):
    B, T, E = h.shape
    H = N_HEADS
    DH = E // H
    BR = 256
    BS = 256
    BQ = 256

    coord_row = coord_1d.reshape(B, 1, T)
    coord_col = coord_1d.reshape(B, T, 1)
    w_qkv = jnp.concatenate([Wq_w, Wk_w, Wv_w], axis=1)
    b_qkv = jnp.concatenate([Wq_b, Wk_b, Wv_b]).reshape(1, 3 * E)
    b_o = Wo_b.reshape(1, E)

    rank_row, rank_col = pl.pallas_call(
        functools.partial(_rank_kernel, T=T, BR=BR),
        grid=(B, T // BR),
        in_specs=[
            pl.BlockSpec((1, T, 1), lambda b, j: (b, 0, 0)),
            pl.BlockSpec((1, 1, BR), lambda b, j: (b, 0, j)),
            pl.BlockSpec((1, 1, T), lambda b, j: (b, 0, 0)),
            pl.BlockSpec((1, BR, 1), lambda b, j: (b, j, 0)),
        ],
        out_specs=[
            pl.BlockSpec((1, 1, BR), lambda b, j: (b, 0, j)),
            pl.BlockSpec((1, BR, 1), lambda b, j: (b, j, 0)),
        ],
        out_shape=[
            jax.ShapeDtypeStruct((B, 1, T), jnp.int32),
            jax.ShapeDtypeStruct((B, T, 1), jnp.int32),
        ],
    )(coord_col, coord_row, coord_row, coord_col)

    depot = rank_row[:, 0, 0]

    qkv, cs_col = pl.pallas_call(
        functools.partial(_gather_qkv_kernel, T=T, BS=BS),
        grid=(B, T // BS),
        in_specs=[
            pl.BlockSpec((1, 1, T), lambda b, i: (b, 0, 0)),
            pl.BlockSpec((1, T, E), lambda b, i: (b, 0, 0)),
            pl.BlockSpec((1, 1, T), lambda b, i: (b, 0, 0)),
            pl.BlockSpec((E, 3 * E), lambda b, i: (0, 0)),
            pl.BlockSpec((1, 3 * E), lambda b, i: (0, 0)),
        ],
        out_specs=[
            pl.BlockSpec((1, BS, 3 * E), lambda b, i: (b, i, 0)),
            pl.BlockSpec((1, BS, 1), lambda b, i: (b, i, 0)),
        ],
        out_shape=[
            jax.ShapeDtypeStruct((B, T, 3 * E), jnp.float32),
            jax.ShapeDtypeStruct((B, T, 1), jnp.float32),
        ],
    )(rank_row, h, coord_row, w_qkv, b_qkv)

    ctx = pl.pallas_call(
        functools.partial(_attn_kernel, T=T, BQ=BQ, H=H, DH=DH),
        grid_spec=pltpu.PrefetchScalarGridSpec(
            num_scalar_prefetch=1,
            grid=(B, T // BQ),
            in_specs=[
                pl.BlockSpec((1, BQ, E), lambda b, i, dref: (b, i, 0)),
                pl.BlockSpec((1, T, E), lambda b, i, dref: (b, 0, 1)),
                pl.BlockSpec((1, T, E), lambda b, i, dref: (b, 0, 2)),
                pl.BlockSpec((1, BQ, 1), lambda b, i, dref: (b, i, 0)),
                pl.BlockSpec((1, T, 1), lambda b, i, dref: (b, 0, 0)),
            ],
            out_specs=pl.BlockSpec((1, BQ, E), lambda b, i, dref: (b, i, 0)),
        ),
        out_shape=jax.ShapeDtypeStruct((B, T, E), jnp.float32),
    )(depot, qkv, qkv, qkv, cs_col, cs_col)

    out = pl.pallas_call(
        functools.partial(_unsort_proj_kernel, T=T, BS=BS),
        grid=(B, T // BS),
        in_specs=[
            pl.BlockSpec((1, BS, 1), lambda b, i: (b, i, 0)),
            pl.BlockSpec((1, T, E), lambda b, i: (b, 0, 0)),
            pl.BlockSpec((E, E), lambda b, i: (0, 0)),
            pl.BlockSpec((1, E), lambda b, i: (0, 0)),
        ],
        out_specs=pl.BlockSpec((1, BS, E), lambda b, i: (b, i, 0)),
        out_shape=jax.ShapeDtypeStruct((B, T, E), jnp.float32),
    )(rank_col, ctx, Wo_w, b_o)

    return out
